# baseline (speedup 1.0000x reference)
import functools

import jax
import jax.numpy as jnp
from jax import lax
from jax.experimental import pallas as pl
from jax.experimental.pallas import tpu as pltpu
from jax.experimental.pallas import tpu_sc as plsc

_L = 16


def _pattern_sc(row_embed, col_embed, H, W, D):
    C = 2 * D
    NW = 32
    ROWS = C // NW
    HW = H * W
    NROW, DROW = row_embed.shape
    NCOL, DCOL = col_embed.shape
    mesh = plsc.VectorSubcoreMesh(core_axis_name="c", subcore_axis_name="s")

    @functools.partial(
        pl.kernel,
        mesh=mesh,
        out_type=jax.ShapeDtypeStruct((C, HW), jnp.float32),
        scratch_types=[
            pltpu.VMEM((NROW * DROW,), jnp.float32),
            pltpu.VMEM((ROWS, HW), jnp.float32),
            pltpu.SemaphoreType.DMA,
        ],
        compiler_params=pltpu.CompilerParams(needs_layout_passes=False),
    )
    def k(row_hbm, col_hbm, out_hbm, tab_v, chunk, sem):
        cid = lax.axis_index("c")
        sid = lax.axis_index("s")
        wid = sid * 2 + cid
        base_c = wid * ROWS
        is_col = base_c < D

        @pl.when(is_col)
        def _():
            pltpu.sync_copy(col_hbm, tab_v)

        @pl.when(jnp.logical_not(is_col))
        def _():
            pltpu.sync_copy(row_hbm, tab_v)

        iota = lax.iota(jnp.int32, _L)

        @pl.when(is_col)
        def _col_half():
            for r in range(ROWS):
                vecs = [
                    plsc.load_gather(
                        tab_v, [(iota + x0) * DCOL + (base_c + r)]
                    )
                    for x0 in range(0, W, _L)
                ]
                for y in range(H):
                    for i, v in enumerate(vecs):
                        chunk[r, pl.ds(y * W + i * _L, _L)] = v

        @pl.when(jnp.logical_not(is_col))
        def _row_half():
            for r in range(ROWS):
                ec = base_c - D + r
                for y in range(H):
                    v = plsc.load_gather(
                        tab_v, [jnp.full((_L,), y * DROW + ec, jnp.int32)]
                    )
                    for x0 in range(0, W, _L):
                        chunk[r, pl.ds(y * W + x0, _L)] = v

        pltpu.async_copy(chunk, out_hbm.at[pl.ds(base_c, ROWS)], sem).wait()

    return k(row_embed.reshape(-1), col_embed.reshape(-1))


def _broadcast_tc(pattern, B, C, H, W):
    def body(pat_ref, out_ref):
        out_ref[...] = pat_ref[...][None]

    return pl.pallas_call(
        body,
        grid=(B,),
        in_specs=[pl.BlockSpec((C, H, W), lambda b: (0, 0, 0))],
        out_specs=pl.BlockSpec((1, C, H, W), lambda b: (b, 0, 0, 0)),
        out_shape=jax.ShapeDtypeStruct((B, C, H, W), jnp.float32),
    )(pattern)


def kernel(x, row_embed, col_embed):
    B = x.shape[0]
    H, W = x.shape[-2], x.shape[-1]
    D = row_embed.shape[-1]
    C = 2 * D
    pattern = _pattern_sc(row_embed, col_embed, H, W, D)
    return _broadcast_tc(pattern.reshape(C, H, W), B, C, H, W)


# speedup vs baseline: 6.4262x; 6.4262x over previous
import functools

import jax
import jax.numpy as jnp
from jax import lax
from jax.experimental import pallas as pl
from jax.experimental.pallas import tpu as pltpu
from jax.experimental.pallas import tpu_sc as plsc

_L = 16


def _pos_embed_sc(row_embed, col_embed, B, H, W, D):
    C = 2 * D
    NW = 32
    assert H == NW and W <= col_embed.shape[0]
    NROW, DROW = row_embed.shape
    mesh = plsc.VectorSubcoreMesh(core_axis_name="c", subcore_axis_name="s")

    @functools.partial(
        pl.kernel,
        mesh=mesh,
        out_type=jax.ShapeDtypeStruct((B, H, W, C), jnp.float32),
        scratch_types=[
            pltpu.VMEM((1, DROW), jnp.float32),
            pltpu.VMEM((1, W, C), jnp.float32),
            pltpu.SemaphoreType.DMA,
        ],
        compiler_params=pltpu.CompilerParams(needs_layout_passes=False),
    )
    def k(row_hbm, col_hbm, out_hbm, rowbuf, plane, sem):
        cid = lax.axis_index("c")
        sid = lax.axis_index("s")
        y = sid * 2 + cid

        cp_col = pltpu.async_copy(
            col_hbm.at[pl.ds(0, W)], plane.at[0, :, pl.ds(0, D)], sem
        )
        cp_row = pltpu.async_copy(row_hbm.at[pl.ds(y, 1)], rowbuf, sem)
        cp_col.wait()
        cp_row.wait()

        vecs = [rowbuf[0, pl.ds(j, _L)] for j in range(0, D, _L)]
        for x in range(W):
            for j, v in enumerate(vecs):
                plane[0, x, pl.ds(D + j * _L, _L)] = v

        copies = [
            pltpu.async_copy(plane, out_hbm.at[b, pl.ds(y, 1)], sem)
            for b in range(B)
        ]
        for cp in copies:
            cp.wait()

    return k(row_embed, col_embed)


def kernel(x, row_embed, col_embed):
    B = x.shape[0]
    H, W = x.shape[-2], x.shape[-1]
    D = row_embed.shape[-1]
    out = _pos_embed_sc(row_embed, col_embed, B, H, W, D)
    return jnp.transpose(out, (0, 3, 1, 2))


# speedup vs baseline: 6.5595x; 1.0208x over previous
import functools

import jax
import jax.numpy as jnp
from jax import lax
from jax.experimental import pallas as pl
from jax.experimental.pallas import tpu as pltpu
from jax.experimental.pallas import tpu_sc as plsc

_L = 16


def _pos_embed_sc(row_embed, col_embed, B, H, W, D):
    C = 2 * D
    NW = 32
    assert H == NW and W <= col_embed.shape[0]
    NROW, DROW = row_embed.shape
    mesh = plsc.VectorSubcoreMesh(core_axis_name="c", subcore_axis_name="s")

    @functools.partial(
        pl.kernel,
        mesh=mesh,
        out_type=jax.ShapeDtypeStruct((B, H, W, C), jnp.float32),
        scratch_types=[
            pltpu.VMEM((1, DROW), jnp.float32),
            pltpu.VMEM((1, W, C), jnp.float32),
            pltpu.SemaphoreType.DMA,
        ],
        compiler_params=pltpu.CompilerParams(needs_layout_passes=False),
    )
    def k(row_hbm, col_hbm, out_hbm, rowbuf, plane, sem):
        cid = lax.axis_index("c")
        sid = lax.axis_index("s")
        y = sid * 2 + cid

        cp_col = pltpu.async_copy(
            col_hbm.at[pl.ds(0, W)], plane.at[0, :, pl.ds(0, D)], sem
        )
        cp_row = pltpu.async_copy(row_hbm.at[pl.ds(y, 1)], rowbuf, sem)
        cp_col.wait()
        cp_row.wait()

        def _store_x(x, _):
            def _store_j(j, _):
                plane[0, x, pl.ds(D + j * _L, _L)] = rowbuf[0, pl.ds(j * _L, _L)]
                return 0

            return lax.fori_loop(0, D // _L, _store_j, 0)

        lax.fori_loop(0, W, _store_x, 0)

        copies = [
            pltpu.async_copy(plane, out_hbm.at[b, pl.ds(y, 1)], sem)
            for b in range(B)
        ]
        for cp in copies:
            cp.wait()

    return k(row_embed, col_embed)


def kernel(x, row_embed, col_embed):
    B = x.shape[0]
    H, W = x.shape[-2], x.shape[-1]
    D = row_embed.shape[-1]
    out = _pos_embed_sc(row_embed, col_embed, B, H, W, D)
    return jnp.transpose(out, (0, 3, 1, 2))


# speedup vs baseline: 6.5830x; 1.0036x over previous
import functools

import jax
import jax.numpy as jnp
from jax import lax
from jax.experimental import pallas as pl
from jax.experimental.pallas import tpu as pltpu
from jax.experimental.pallas import tpu_sc as plsc

_L = 16


def _pos_embed_sc(row_embed, col_embed, B, H, W, D):
    C = 2 * D
    NW = 32
    assert H == NW and W <= col_embed.shape[0]
    NROW, DROW = row_embed.shape
    mesh = plsc.VectorSubcoreMesh(core_axis_name="c", subcore_axis_name="s")

    @functools.partial(
        pl.kernel,
        mesh=mesh,
        out_type=jax.ShapeDtypeStruct((B, H, W, C), jnp.float32),
        scratch_types=[
            pltpu.VMEM((1, DROW), jnp.float32),
            pltpu.VMEM((1, W, C), jnp.float32),
            pltpu.SemaphoreType.DMA,
        ],
        compiler_params=pltpu.CompilerParams(needs_layout_passes=False),
    )
    def k(row_hbm, col_hbm, out_hbm, rowbuf, plane, sem):
        cid = lax.axis_index("c")
        sid = lax.axis_index("s")
        y = sid * 2 + cid

        cp_col = pltpu.async_copy(
            col_hbm.at[pl.ds(0, W)], plane.at[0, :, pl.ds(0, D)], sem
        )
        cp_row = pltpu.async_copy(row_hbm.at[pl.ds(y, 1)], rowbuf, sem)
        cp_col.wait()
        cp_row.wait()

        def _store_x(x, _):
            for j in range(D // _L):
                plane[0, x, pl.ds(D + j * _L, _L)] = rowbuf[0, pl.ds(j * _L, _L)]
            return 0

        lax.fori_loop(0, W, _store_x, 0)

        copies = [
            pltpu.async_copy(plane, out_hbm.at[b, pl.ds(y, 1)], sem)
            for b in range(B)
        ]
        for cp in copies:
            cp.wait()

    return k(row_embed, col_embed)


def kernel(x, row_embed, col_embed):
    B = x.shape[0]
    H, W = x.shape[-2], x.shape[-1]
    D = row_embed.shape[-1]
    out = _pos_embed_sc(row_embed, col_embed, B, H, W, D)
    return jnp.transpose(out, (0, 3, 1, 2))


# speedup vs baseline: 6.8094x; 1.0344x over previous
import functools

import jax
import jax.numpy as jnp
from jax import lax
from jax.experimental import pallas as pl
from jax.experimental.pallas import tpu as pltpu
from jax.experimental.pallas import tpu_sc as plsc

_L = 16


def _pos_embed_sc(row_embed, col_embed, B, H, W, D):
    C = 2 * D
    NW = 32
    assert H == NW and W <= col_embed.shape[0]
    NROW, DROW = row_embed.shape
    mesh = plsc.VectorSubcoreMesh(core_axis_name="c", subcore_axis_name="s")

    @functools.partial(
        pl.kernel,
        mesh=mesh,
        out_type=jax.ShapeDtypeStruct((B, H, W, C), jnp.float32),
        scratch_types=[
            pltpu.VMEM((1, DROW), jnp.float32),
            pltpu.VMEM((1, W, C), jnp.float32),
            pltpu.SemaphoreType.DMA,
        ],
        compiler_params=pltpu.CompilerParams(needs_layout_passes=False),
    )
    def k(row_hbm, col_hbm, out_hbm, rowbuf, plane, sem):
        cid = lax.axis_index("c")
        sid = lax.axis_index("s")
        y = sid * 2 + cid

        cp_col = pltpu.async_copy(
            col_hbm.at[pl.ds(0, W)], plane.at[0, :, pl.ds(0, D)], sem
        )
        cp_row = pltpu.async_copy(row_hbm.at[pl.ds(y, 1)], rowbuf, sem)
        cp_row.wait()

        def _store_x(x, _):
            for j in range(D // _L):
                plane[0, x, pl.ds(D + j * _L, _L)] = rowbuf[0, pl.ds(j * _L, _L)]
            return 0

        lax.fori_loop(0, W, _store_x, 0)
        cp_col.wait()

        copies = [
            pltpu.async_copy(
                plane, out_hbm.at[(y + b) % B, pl.ds(y, 1)], sem
            )
            for b in range(B)
        ]
        for cp in copies:
            cp.wait()

    return k(row_embed, col_embed)


def kernel(x, row_embed, col_embed):
    B = x.shape[0]
    H, W = x.shape[-2], x.shape[-1]
    D = row_embed.shape[-1]
    out = _pos_embed_sc(row_embed, col_embed, B, H, W, D)
    return jnp.transpose(out, (0, 3, 1, 2))
